# baseline (speedup 1.0000x reference)
import functools

import jax
import jax.numpy as jnp
from jax import lax
from jax.experimental import pallas as pl
from jax.experimental.pallas import tpu as pltpu
from jax.experimental.pallas import tpu_sc as plsc

H = 4
C = 32
D = 128
B = 128
NS = 16
NC = 2
R = 400
_SC_PARAMS = pltpu.CompilerParams(needs_layout_passes=False)
_LANE = None


def _cdiv(a, b):
    return (a + b - 1) // b


def _prep_body(cx_ref, wu_ref, bu_ref, fx_ref, wi_ref, bi_ref,
               wl1_ref, bl1_ref, wr1_ref, br1_ref, wr2_ref, br2_ref,
               xl1_ref, xr1_ref, xr2_ref):
    ux = jnp.dot(cx_ref[...], wu_ref[...],
                 preferred_element_type=jnp.float32) + bu_ref[...]
    ix = fx_ref[...] * wi_ref[...] + bi_ref[...]
    xl1_ref[...] = jnp.dot(ux, wl1_ref[...],
                           preferred_element_type=jnp.float32) + bl1_ref[...]
    xr1_ref[...] = jnp.dot(ix, wr1_ref[...],
                           preferred_element_type=jnp.float32) + br1_ref[...]
    xr2_ref[...] = jnp.dot(ux, wr2_ref[...],
                           preferred_element_type=jnp.float32) + br2_ref[...]


def _prep(cx, Wu, bu, fx, Wi, bi, Wl1, bl1, Wr1, br1, Wr2, br2):
    n = cx.shape[0]
    k = cx.shape[1]
    grid = n // R
    return pl.pallas_call(
        _prep_body,
        grid=(grid,),
        in_specs=[
            pl.BlockSpec((R, k), lambda i: (i, 0)),
            pl.BlockSpec((k, 32), lambda i: (0, 0)),
            pl.BlockSpec((1, 32), lambda i: (0, 0)),
            pl.BlockSpec((R, 1), lambda i: (i, 0)),
            pl.BlockSpec((1, 32), lambda i: (0, 0)),
            pl.BlockSpec((1, 32), lambda i: (0, 0)),
            pl.BlockSpec((32, D), lambda i: (0, 0)),
            pl.BlockSpec((1, D), lambda i: (0, 0)),
            pl.BlockSpec((32, D), lambda i: (0, 0)),
            pl.BlockSpec((1, D), lambda i: (0, 0)),
            pl.BlockSpec((32, D), lambda i: (0, 0)),
            pl.BlockSpec((1, D), lambda i: (0, 0)),
        ],
        out_specs=[
            pl.BlockSpec((R, D), lambda i: (i, 0)),
            pl.BlockSpec((R, D), lambda i: (i, 0)),
            pl.BlockSpec((R, D), lambda i: (i, 0)),
        ],
        out_shape=[jax.ShapeDtypeStruct((n, D), jnp.float32)] * 3,
    )(cx, Wu, bu.reshape(1, 32), fx, Wi.reshape(1, 32), bi.reshape(1, 32),
      Wl1, bl1.reshape(1, D), Wr1, br1.reshape(1, D), Wr2, br2.reshape(1, D))


def _attrsum_body(at_ref, out_ref):
    i = pl.program_id(0)

    @pl.when(i == 0)
    def _():
        out_ref[...] = jnp.zeros_like(out_ref)

    s3 = jnp.sum(at_ref[...], axis=1)
    pad = jnp.concatenate([s3, jnp.zeros((125,), jnp.float32)])
    out_ref[...] += pad.reshape(1, D)


def _attrsum(attrT):
    e_pad = attrT.shape[1]
    blk = 6272
    grid = e_pad // blk
    return pl.pallas_call(
        _attrsum_body,
        grid=(grid,),
        in_specs=[pl.BlockSpec((3, blk), lambda i: (0, i))],
        out_specs=pl.BlockSpec((1, D), lambda i: (0, 0)),
        out_shape=jax.ShapeDtypeStruct((1, D), jnp.float32),
    )(attrT)


def _p1_body(has_attr, n_blocks, ept,
             xl_hbm, xr_hbm, s_hbm, d_hbm, a0_hbm, a1_hbm, a2_hbm,
             we_hbm, att_hbm,
             ea0_hbm, ea1_hbm, ea2_hbm, ea3_hbm,
             sidxA, didxA, a0A, a1A, a2A, rlA, rrA,
             sidxB, didxB, a0B, a1B, a2B, rlB, rrB,
             we_v, att_v, ea0_v, ea1_v, ea2_v, ea3_v,
             semLA, semLB, semGA, semGB, semW):
    core = lax.axis_index("c")
    sub = lax.axis_index("s")
    wid = sub * NC + core
    pltpu.sync_copy(att_hbm, att_v)
    if has_attr:
        pltpu.sync_copy(we_hbm, we_v)
    lane = lax.iota(jnp.int32, 16)
    ea_outs = (ea0_hbm, ea1_hbm, ea2_hbm, ea3_hbm)
    ea_vs = (ea0_v, ea1_v, ea2_v, ea3_v)
    a_hbms = (a0_hbm, a1_hbm, a2_hbm)
    bufs = (
        (sidxA, didxA, (a0A, a1A, a2A), rlA, rrA, semLA, semGA),
        (sidxB, didxB, (a0B, a1B, a2B), rlB, rrB, semLB, semGB),
    )

    def fire_loads(bi, bs):
        sidx, didx, avs, _rl, _rr, semL, _semG = bs
        e0 = wid * ept + bi * B
        pltpu.async_copy(s_hbm.at[pl.ds(e0, B)], sidx, semL)
        pltpu.async_copy(d_hbm.at[pl.ds(e0, B)], didx, semL)
        if has_attr:
            for k in range(3):
                pltpu.async_copy(a_hbms[k].at[pl.ds(e0, B)], avs[k], semL)

    def wait_loads(bi, bs):
        sidx, didx, avs, _rl, _rr, semL, _semG = bs
        e0 = wid * ept + bi * B
        pltpu.make_async_copy(s_hbm.at[pl.ds(e0, B)], sidx, semL).wait()
        pltpu.make_async_copy(d_hbm.at[pl.ds(e0, B)], didx, semL).wait()
        if has_attr:
            for k in range(3):
                pltpu.make_async_copy(a_hbms[k].at[pl.ds(e0, B)], avs[k],
                                      semL).wait()

    def fire_gathers(bs):
        sidx, didx, _avs, rl, rr, _semL, semG = bs
        pltpu.async_copy(xl_hbm.at[sidx], rl, semG)
        pltpu.async_copy(xr_hbm.at[didx], rr, semG)

    def wait_gathers(bs):
        sidx, didx, _avs, rl, rr, _semL, semG = bs
        pltpu.make_async_copy(xl_hbm.at[sidx], rl, semG).wait()
        pltpu.make_async_copy(xr_hbm.at[didx], rr, semG).wait()

    fire_loads(0, bufs[0])
    wait_loads(0, bufs[0])
    fire_gathers(bufs[0])

    def phase(i, bs, bsn, last_guard):
        _sidx, _didx, avs, rows_l, rows_r, _semL, _semG = bs
        e0 = wid * ept + i * B
        if last_guard is None:
            fire_loads(i + 1, bsn)
        else:
            @pl.when(last_guard)
            def _fl():
                fire_loads(i + 1, bsn)
        wait_gathers(bs)

        def group(g, carry):
            attr_g = None
            if has_attr:
                attr_g = [avs[k][pl.ds(g * 16, 16)] for k in range(3)]
            att_r = [att_v[pl.ds(v * 16, 16)] for v in range(8)]
            we_r = None
            if has_attr:
                we_r = [[we_v[k, pl.ds(v * 16, 16)] for v in range(8)]
                        for k in range(3)]

            def edge(j, carry2):
                e = g * 16 + j
                if has_attr:
                    aks = [jnp.sum(jnp.where(lane == j, attr_g[k], 0.0))
                           for k in range(3)]
                cons = []
                for v in range(8):
                    s = (rows_l[e, pl.ds(v * 16, 16)]
                         + rows_r[e, pl.ds(v * 16, 16)])
                    if has_attr:
                        s = s + aks[0] * we_r[0][v] + aks[1] * we_r[1][v] \
                            + aks[2] * we_r[2][v]
                    l = jnp.maximum(s, 0.2 * s)
                    cons.append(l * att_r[v])
                out = []
                for h in range(4):
                    sh = cons[2 * h] + cons[2 * h + 1]
                    alpha = jnp.sum(sh)
                    ev = jnp.exp(jnp.full((16,), alpha, jnp.float32))
                    out.append(jnp.where(lane == j, ev, carry2[h]))
                return tuple(out)

            z = jnp.zeros((16,), jnp.float32)
            eas = lax.fori_loop(0, 16, edge, (z, z, z, z))
            for h in range(4):
                ea_vs[h][pl.ds(g * 16, 16)] = eas[h]
            return carry

        lax.fori_loop(0, B // 16, group, 0)

        def fire_next_gather():
            wait_loads(i + 1, bsn)
            fire_gathers(bsn)
        if last_guard is None:
            fire_next_gather()
        else:
            @pl.when(last_guard)
            def _fg():
                fire_next_gather()

        for h in range(4):
            pltpu.async_copy(ea_vs[h], ea_outs[h].at[pl.ds(e0, B)], semW)
        for h in range(4):
            pltpu.make_async_copy(ea_vs[h], ea_outs[h].at[pl.ds(e0, B)],
                                  semW).wait()

    def pair(p, carry):
        phase(2 * p, bufs[0], bufs[1], None)
        phase(2 * p + 1, bufs[1], bufs[0], p < n_blocks // 2 - 1)
        return carry

    lax.fori_loop(0, n_blocks // 2, pair, 0)


def _p1(xl, xr, s_idx, d_idx, attrs, We, att_flat):
    e_pad = s_idx.shape[0]
    ept = e_pad // (NC * NS)
    n_blocks = ept // B
    has_attr = attrs is not None
    mesh = plsc.VectorSubcoreMesh(core_axis_name="c", subcore_axis_name="s")
    if not has_attr:
        attrs = [jnp.zeros((128,), jnp.float32)] * 3
        We = jnp.zeros((3, D), jnp.float32)
    kern = functools.partial(
        pl.kernel,
        mesh=mesh,
        compiler_params=_SC_PARAMS,
        out_type=tuple(jax.ShapeDtypeStruct((e_pad,), jnp.float32)
                       for _ in range(4)),
        scratch_types=[
            pltpu.VMEM((B,), jnp.int32),
            pltpu.VMEM((B,), jnp.int32),
            pltpu.VMEM((B,), jnp.float32),
            pltpu.VMEM((B,), jnp.float32),
            pltpu.VMEM((B,), jnp.float32),
            pltpu.VMEM((B, D), jnp.float32),
            pltpu.VMEM((B, D), jnp.float32),
            pltpu.VMEM((B,), jnp.int32),
            pltpu.VMEM((B,), jnp.int32),
            pltpu.VMEM((B,), jnp.float32),
            pltpu.VMEM((B,), jnp.float32),
            pltpu.VMEM((B,), jnp.float32),
            pltpu.VMEM((B, D), jnp.float32),
            pltpu.VMEM((B, D), jnp.float32),
            pltpu.VMEM((3, D), jnp.float32),
            pltpu.VMEM((D,), jnp.float32),
            pltpu.VMEM((B,), jnp.float32),
            pltpu.VMEM((B,), jnp.float32),
            pltpu.VMEM((B,), jnp.float32),
            pltpu.VMEM((B,), jnp.float32),
            pltpu.SemaphoreType.DMA,
            pltpu.SemaphoreType.DMA,
            pltpu.SemaphoreType.DMA,
            pltpu.SemaphoreType.DMA,
            pltpu.SemaphoreType.DMA,
        ],
    )(functools.partial(_p1_body, has_attr, n_blocks, ept))
    return kern(xl, xr, s_idx, d_idx, attrs[0], attrs[1], attrs[2],
                We, att_flat)


def _p2_body(n_real_edges, n_blocks, ept, rc, ncper,
             xl_hbm, key_hbm, g_hbm, ea0_hbm, ea1_hbm, ea2_hbm, ea3_hbm,
             out_hbm, den_hbm,
             kidxA, sidxA, ea0A, ea1A, ea2A, ea3A, dlA, dpA, rowsA,
             kidxB, sidxB, ea0B, ea1B, ea2B, ea3B, dlB, dpB, rowsB,
             easc, zbuf, acc, dacc,
             semLA, semLB, semGA, semGB):
    ea_hbms = (ea0_hbm, ea1_hbm, ea2_hbm, ea3_hbm)
    bufs = (
        (kidxA, sidxA, (ea0A, ea1A, ea2A, ea3A), dlA, dpA, rowsA,
         semLA, semGA),
        (kidxB, sidxB, (ea0B, ea1B, ea2B, ea3B), dlB, dpB, rowsB,
         semLB, semGB),
    )
    core = lax.axis_index("c")
    sub = lax.axis_index("s")
    rpt = rc // NS
    rc8 = rc // 16
    rc8p = rc8 + 32
    lane = lax.iota(jnp.int32, 16)

    def zrow(r, _):
        for v in range(D // 16):
            zbuf[r, pl.ds(v * 16, 16)] = jnp.zeros((16,), jnp.float32)
        return _
    lax.fori_loop(0, 32, zrow, 0)

    def zrowe(r, _):
        for v in range(D // 16):
            easc[r, pl.ds(v * 16, 16)] = jnp.zeros((16,), jnp.float32)
        return _
    lax.fori_loop(0, B, zrowe, 0)

    def fire_loads(bi, bs):
        kidx, sidx, eav, _dl, _dp, _rows, semL, _semG = bs
        e0 = sub * ept + bi * B
        pltpu.async_copy(key_hbm.at[pl.ds(e0, B)], kidx, semL)
        pltpu.async_copy(g_hbm.at[pl.ds(e0, B)], sidx, semL)
        for h in range(4):
            pltpu.async_copy(ea_hbms[h].at[pl.ds(e0, B)], eav[h], semL)

    def wait_loads(bi, bs):
        kidx, sidx, eav, _dl, _dp, _rows, semL, _semG = bs
        e0 = sub * ept + bi * B
        pltpu.make_async_copy(key_hbm.at[pl.ds(e0, B)], kidx, semL).wait()
        pltpu.make_async_copy(g_hbm.at[pl.ds(e0, B)], sidx, semL).wait()
        for h in range(4):
            pltpu.make_async_copy(ea_hbms[h].at[pl.ds(e0, B)], eav[h],
                                  semL).wait()

    for cc in range(ncper):
        chunk = core * ncper + cc
        r0 = chunk * rc
        for z in range(rpt // 32):
            pltpu.sync_copy(zbuf, acc.at[pl.ds(sub * rpt + z * 32, 32)])
        for z in range(rc8p // 512):
            pltpu.sync_copy(zbuf, dacc.at[pl.ds(z * 512 + sub * 32, 32)])
        krem = (rc8p % 512) // 32

        @pl.when(sub < krem)
        def _zd():
            pltpu.sync_copy(
                zbuf, dacc.at[pl.ds((rc8p // 512) * 512 + sub * 32, 32)])
        plsc.subcore_barrier()

        fire_loads(0, bufs[0])
        wait_loads(0, bufs[0])
        pltpu.async_copy(xl_hbm.at[bufs[0][1]], bufs[0][5], bufs[0][7])

        def phase(i, bs, bsn, p, last_guard):
            kidx, sidx, eav, dl_v, dp_v, rows, _semL, semG = bs
            e0 = sub * ept + i * B
            if last_guard is None:
                fire_loads(i + 1, bsn)
            else:
                @pl.when(last_guard)
                def _fl():
                    fire_loads(i + 1, bsn)
            pltpu.make_async_copy(xl_hbm.at[sidx], rows, semG).wait()

            def group(g, carry):
                kv = kidx[pl.ds(g * 16, 16)]
                dl = kv - r0
                eg = e0 + g * 16
                m = (dl >= 0) & (dl < rc) & ((eg + lane) < n_real_edges)
                dl_v[pl.ds(g * 16, 16)] = jnp.where(m, dl, rc)
                dp_v[pl.ds(g * 16, 16)] = jnp.where(m, dl >> 4, rc8)
                fm = jnp.where(m, 1.0, 0.0)
                col = (dl & 15) * 8
                ridx = g * 16 + lane
                eah = [eav[h][pl.ds(g * 16, 16)] * fm for h in range(4)]

                @pl.when(jnp.any(m))
                def _scale():
                    for h in range(4):
                        plsc.store_scatter(easc, [ridx, col + h], eah[h])
                    for c in range(D):
                        cidx = jnp.full((16,), c, jnp.int32)
                        cv = plsc.load_gather(rows, [ridx, cidx])
                        plsc.store_scatter(rows, [ridx, cidx],
                                           cv * eah[c // 32])
                return carry

            lax.fori_loop(0, B // 16, group, 0)

            def fire_next_gather():
                wait_loads(i + 1, bsn)
                pltpu.async_copy(xl_hbm.at[bsn[1]], bsn[5], bsn[7])
            if last_guard is None:
                fire_next_gather()
            else:
                @pl.when(last_guard)
                def _fg():
                    fire_next_gather()

            pltpu.sync_copy(rows, acc.at[dl_v], add=True)
            pltpu.sync_copy(easc, dacc.at[dp_v], add=True)

            def unscatter(g, carry):
                kv = kidx[pl.ds(g * 16, 16)]
                dl = kv - r0
                eg = e0 + g * 16
                m = (dl >= 0) & (dl < rc) & ((eg + lane) < n_real_edges)
                col = (dl & 15) * 8
                ridx = g * 16 + lane
                zv = jnp.zeros((16,), jnp.float32)

                @pl.when(jnp.any(m))
                def _zs():
                    for h in range(4):
                        plsc.store_scatter(easc, [ridx, col + h], zv)
                return carry

            lax.fori_loop(0, B // 16, unscatter, 0)

        def pair(p, carry):
            phase(2 * p, bufs[0], bufs[1], p, None)
            phase(2 * p + 1, bufs[1], bufs[0], p, p < n_blocks // 2 - 1)
            return carry

        lax.fori_loop(0, n_blocks // 2, pair, 0)
        plsc.subcore_barrier()
        pltpu.sync_copy(acc.at[pl.ds(sub * rpt, rpt)],
                        out_hbm.at[pl.ds(r0 + sub * rpt, rpt)])
        for z in range(rc8 // 512):
            pltpu.sync_copy(
                dacc.at[pl.ds(z * 512 + sub * 32, 32)],
                den_hbm.at[pl.ds(chunk * rc8 + z * 512 + sub * 32, 32)])
        krem2 = (rc8 % 512) // 32

        @pl.when(sub < krem2)
        def _rd():
            off = (rc8 // 512) * 512 + sub * 32
            pltpu.sync_copy(dacc.at[pl.ds(off, 32)],
                            den_hbm.at[pl.ds(chunk * rc8 + off, 32)])
        plsc.subcore_barrier()


def _p2(xl, key_idx, gather_idx, ea, n_real_edges, n_pad, rc, ncper):
    e_pad = key_idx.shape[0]
    ept = e_pad // NS
    n_blocks = ept // B
    rc8 = rc // 16
    mesh = plsc.VectorSubcoreMesh(core_axis_name="c", subcore_axis_name="s")
    kern = functools.partial(
        pl.kernel,
        mesh=mesh,
        compiler_params=_SC_PARAMS,
        out_type=(jax.ShapeDtypeStruct((n_pad, D), jnp.float32),
                  jax.ShapeDtypeStruct((n_pad // 16, D), jnp.float32)),
        scratch_types=[
            pltpu.VMEM((B,), jnp.int32),
            pltpu.VMEM((B,), jnp.int32),
            pltpu.VMEM((B,), jnp.float32),
            pltpu.VMEM((B,), jnp.float32),
            pltpu.VMEM((B,), jnp.float32),
            pltpu.VMEM((B,), jnp.float32),
            pltpu.VMEM((B,), jnp.int32),
            pltpu.VMEM((B,), jnp.int32),
            pltpu.VMEM((B, D), jnp.float32),
            pltpu.VMEM((B,), jnp.int32),
            pltpu.VMEM((B,), jnp.int32),
            pltpu.VMEM((B,), jnp.float32),
            pltpu.VMEM((B,), jnp.float32),
            pltpu.VMEM((B,), jnp.float32),
            pltpu.VMEM((B,), jnp.float32),
            pltpu.VMEM((B,), jnp.int32),
            pltpu.VMEM((B,), jnp.int32),
            pltpu.VMEM((B, D), jnp.float32),
            pltpu.VMEM((B, D), jnp.float32),
            pltpu.VMEM((32, D), jnp.float32),
            pltpu.VMEM_SHARED((rc + 8, D), jnp.float32),
            pltpu.VMEM_SHARED((rc8 + 32, D), jnp.float32),
            pltpu.SemaphoreType.DMA,
            pltpu.SemaphoreType.DMA,
            pltpu.SemaphoreType.DMA,
            pltpu.SemaphoreType.DMA,
        ],
    )(functools.partial(_p2_body, n_real_edges, n_blocks, ept, rc, ncper))
    return kern(xl, key_idx, gather_idx, ea[0], ea[1], ea[2], ea[3])


def _fin1_body(n_real_edges,
               xl_ref, xr_ref, osc_ref, dsc_ref, asum_ref, we_ref, att_ref,
               bias_ref, wl2_ref, bl2_ref, xl2_ref):
    mean3 = asum_ref[0, 0:3] / n_real_edges
    eproj = jnp.sum(mean3[:, None] * we_ref[...], axis=0)
    xl = xl_ref[...]
    s = xl + xr_ref[...] + eproj[None, :]
    l = jnp.maximum(s, 0.2 * s)
    alpha = jnp.sum((l * att_ref[...]).reshape(-1, H, C), axis=-1)
    ea = jnp.exp(alpha)
    den = dsc_ref[:, 0:4] + ea + 1e-16
    eae = jnp.broadcast_to(ea[:, :, None], (ea.shape[0], H, C)).reshape(-1, D)
    dene = jnp.broadcast_to(den[:, :, None],
                            (den.shape[0], H, C)).reshape(-1, D)
    item_h = jnp.maximum((osc_ref[...] + eae * xl) / dene + bias_ref[...],
                         0.0)
    xl2_ref[...] = jnp.dot(item_h, wl2_ref[...],
                           preferred_element_type=jnp.float32) + bl2_ref[...]


def _fin1(n, n_pad, n_real_edges, xl1, xr1, osc, dsc, asum, We1, att1f,
          bias1, Wl2, bl2):
    grid = n // R
    return pl.pallas_call(
        functools.partial(_fin1_body, n_real_edges),
        grid=(grid,),
        in_specs=[
            pl.BlockSpec((R, D), lambda i: (i, 0)),
            pl.BlockSpec((R, D), lambda i: (i, 0)),
            pl.BlockSpec((R, D), lambda i: (i, 0)),
            pl.BlockSpec((R, 8), lambda i: (i, 0)),
            pl.BlockSpec((1, D), lambda i: (0, 0)),
            pl.BlockSpec((3, D), lambda i: (0, 0)),
            pl.BlockSpec((1, D), lambda i: (0, 0)),
            pl.BlockSpec((1, D), lambda i: (0, 0)),
            pl.BlockSpec((D, D), lambda i: (0, 0)),
            pl.BlockSpec((1, D), lambda i: (0, 0)),
        ],
        out_specs=pl.BlockSpec((R, D), lambda i: (i, 0)),
        out_shape=jax.ShapeDtypeStruct((n, D), jnp.float32),
    )(xl1, xr1, osc, dsc, asum, We1, att1f, bias1, Wl2, bl2.reshape(1, D))


def _fin2_body(xl_ref, xr_ref, osc_ref, dsc_ref, att_ref, bias_ref,
               wp1_ref, bp1_ref, wp2_ref, bp2_ref, uh_ref, z_ref):
    xl = xl_ref[...]
    s = xl + xr_ref[...]
    l = jnp.maximum(s, 0.2 * s)
    alpha = jnp.sum((l * att_ref[...]).reshape(-1, H, C), axis=-1)
    ea = jnp.exp(alpha)
    den = dsc_ref[:, 0:4] + ea + 1e-16
    eae = jnp.broadcast_to(ea[:, :, None], (ea.shape[0], H, C)).reshape(-1, D)
    dene = jnp.broadcast_to(den[:, :, None],
                            (den.shape[0], H, C)).reshape(-1, D)
    uh = (osc_ref[...] + eae * xl) / dene + bias_ref[...]
    uh_ref[...] = uh
    hh = jnp.maximum(
        jnp.dot(uh, wp1_ref[...], preferred_element_type=jnp.float32)
        + bp1_ref[...], 0.0)
    z_ref[...] = jnp.dot(hh, wp2_ref[...],
                         preferred_element_type=jnp.float32) + bp2_ref[...]


def _fin2(n, xl2, xr2, osc, dsc, att2f, bias2, Wp1, bp1, Wp2, bp2):
    grid = n // R
    return pl.pallas_call(
        _fin2_body,
        grid=(grid,),
        in_specs=[
            pl.BlockSpec((R, D), lambda i: (i, 0)),
            pl.BlockSpec((R, D), lambda i: (i, 0)),
            pl.BlockSpec((R, D), lambda i: (i, 0)),
            pl.BlockSpec((R, 8), lambda i: (i, 0)),
            pl.BlockSpec((1, D), lambda i: (0, 0)),
            pl.BlockSpec((1, D), lambda i: (0, 0)),
            pl.BlockSpec((D, D), lambda i: (0, 0)),
            pl.BlockSpec((1, D), lambda i: (0, 0)),
            pl.BlockSpec((D, D), lambda i: (0, 0)),
            pl.BlockSpec((1, D), lambda i: (0, 0)),
        ],
        out_specs=[
            pl.BlockSpec((R, D), lambda i: (i, 0)),
            pl.BlockSpec((R, D), lambda i: (i, 0)),
        ],
        out_shape=[jax.ShapeDtypeStruct((n, D), jnp.float32)] * 2,
    )(xl2, xr2, osc, dsc, att2f, bias2, Wp1, bp1.reshape(1, D), Wp2,
      bp2.reshape(1, D))


def kernel(customer_x, fund_x, edge_index, edge_attr, Wu, bu, Wi, bi,
           Wl1, bl1, Wr1, br1, att1, We1, bias1,
           Wl2, bl2, Wr2, br2, att2, bias2, Wp1, bp1, Wp2, bp2):
    n = customer_x.shape[0]
    e = edge_index.shape[1]
    e_pad = _cdiv(e, NC * NS * B) * NC * NS * B
    ncper = 3
    rc = _cdiv(_cdiv(n, 2 * ncper), NS * 32) * NS * 32
    n_pad = 2 * ncper * rc

    src = edge_index[0]
    dst = edge_index[1]
    srcp = jnp.zeros((e_pad,), jnp.int32).at[:e].set(src)
    dstp = jnp.zeros((e_pad,), jnp.int32).at[:e].set(dst)
    attrT = jnp.zeros((3, e_pad), jnp.float32).at[:, :e].set(edge_attr.T)
    attrs = [attrT[0], attrT[1], attrT[2]]
    att1f = att1.reshape(1, D)
    att2f = att2.reshape(1, D)

    xl1, xr1, xr2 = _prep(customer_x, Wu, bu, fund_x, Wi, bi,
                          Wl1, bl1, Wr1, br1, Wr2, br2)
    asum = _attrsum(attrT)

    ea1 = _p1(xl1, xr1, srcp, dstp, attrs, We1, att1f.reshape(D))
    osc1, dp1 = _p2(xl1, dstp, srcp, ea1, e, n_pad, rc, ncper)
    dsc1 = dp1.reshape(n_pad, 8)
    xl2 = _fin1(n, n_pad, e, xl1, xr1, osc1, dsc1, asum, We1, att1f,
                bias1.reshape(1, D), Wl2, bl2)

    ea2 = _p1(xl2, xr2, dstp, srcp, None, None, att2f.reshape(D))
    osc2, dp2 = _p2(xl2, srcp, dstp, ea2, e, n_pad, rc, ncper)
    dsc2 = dp2.reshape(n_pad, 8)
    user_h, z = _fin2(n, xl2, xr2, osc2, dsc2, att2f, bias2.reshape(1, D),
                      Wp1, bp1, Wp2, bp2)
    return (user_h, z)


# speedup vs baseline: 4.0860x; 4.0860x over previous
import functools

import jax
import jax.numpy as jnp
from jax import lax
from jax.experimental import pallas as pl
from jax.experimental.pallas import tpu as pltpu
from jax.experimental.pallas import tpu_sc as plsc

H = 4
C = 32
D = 128
B = 128
NS = 16
NC = 2
R = 400
_SC_PARAMS = pltpu.CompilerParams(needs_layout_passes=False)
_LANE = None


def _cdiv(a, b):
    return (a + b - 1) // b


def _prep_body(cx_ref, wu_ref, bu_ref, fx_ref, wi_ref, bi_ref,
               wl1_ref, bl1_ref, wr1_ref, br1_ref, wr2_ref, br2_ref,
               xl1_ref, xr1_ref, xr2_ref):
    ux = jnp.dot(cx_ref[...], wu_ref[...],
                 preferred_element_type=jnp.float32) + bu_ref[...]
    ix = fx_ref[...] * wi_ref[...] + bi_ref[...]
    xl1_ref[...] = jnp.dot(ux, wl1_ref[...],
                           preferred_element_type=jnp.float32) + bl1_ref[...]
    xr1_ref[...] = jnp.dot(ix, wr1_ref[...],
                           preferred_element_type=jnp.float32) + br1_ref[...]
    xr2_ref[...] = jnp.dot(ux, wr2_ref[...],
                           preferred_element_type=jnp.float32) + br2_ref[...]


def _prep(cx, Wu, bu, fx, Wi, bi, Wl1, bl1, Wr1, br1, Wr2, br2):
    n = cx.shape[0]
    k = cx.shape[1]
    grid = n // R
    return pl.pallas_call(
        _prep_body,
        grid=(grid,),
        in_specs=[
            pl.BlockSpec((R, k), lambda i: (i, 0)),
            pl.BlockSpec((k, 32), lambda i: (0, 0)),
            pl.BlockSpec((1, 32), lambda i: (0, 0)),
            pl.BlockSpec((R, 1), lambda i: (i, 0)),
            pl.BlockSpec((1, 32), lambda i: (0, 0)),
            pl.BlockSpec((1, 32), lambda i: (0, 0)),
            pl.BlockSpec((32, D), lambda i: (0, 0)),
            pl.BlockSpec((1, D), lambda i: (0, 0)),
            pl.BlockSpec((32, D), lambda i: (0, 0)),
            pl.BlockSpec((1, D), lambda i: (0, 0)),
            pl.BlockSpec((32, D), lambda i: (0, 0)),
            pl.BlockSpec((1, D), lambda i: (0, 0)),
        ],
        out_specs=[
            pl.BlockSpec((R, D), lambda i: (i, 0)),
            pl.BlockSpec((R, D), lambda i: (i, 0)),
            pl.BlockSpec((R, D), lambda i: (i, 0)),
        ],
        out_shape=[jax.ShapeDtypeStruct((n, D), jnp.float32)] * 3,
    )(cx, Wu, bu.reshape(1, 32), fx, Wi.reshape(1, 32), bi.reshape(1, 32),
      Wl1, bl1.reshape(1, D), Wr1, br1.reshape(1, D), Wr2, br2.reshape(1, D))


def _attrsum_body(at_ref, out_ref):
    i = pl.program_id(0)

    @pl.when(i == 0)
    def _():
        out_ref[...] = jnp.zeros_like(out_ref)

    s3 = jnp.sum(at_ref[...], axis=1)
    pad = jnp.concatenate([s3, jnp.zeros((125,), jnp.float32)])
    out_ref[...] += pad.reshape(1, D)


def _attrsum(attrT):
    e_pad = attrT.shape[1]
    blk = 6272
    grid = e_pad // blk
    return pl.pallas_call(
        _attrsum_body,
        grid=(grid,),
        in_specs=[pl.BlockSpec((3, blk), lambda i: (0, i))],
        out_specs=pl.BlockSpec((1, D), lambda i: (0, 0)),
        out_shape=jax.ShapeDtypeStruct((1, D), jnp.float32),
    )(attrT)


def _p1_body(has_attr, n_blocks, ept,
             xl_hbm, xr_hbm, s_hbm, d_hbm, a0_hbm, a1_hbm, a2_hbm,
             we_hbm, att_hbm,
             ea0_hbm, ea1_hbm, ea2_hbm, ea3_hbm,
             sidxA, didxA, a0A, a1A, a2A, rlA, rrA,
             sidxB, didxB, a0B, a1B, a2B, rlB, rrB,
             we_v, att_v, ea0_v, ea1_v, ea2_v, ea3_v,
             semLA, semLB, semGA, semGB, semW):
    core = lax.axis_index("c")
    sub = lax.axis_index("s")
    wid = sub * NC + core
    pltpu.sync_copy(att_hbm, att_v)
    if has_attr:
        pltpu.sync_copy(we_hbm, we_v)
    lane = lax.iota(jnp.int32, 16)
    ea_outs = (ea0_hbm, ea1_hbm, ea2_hbm, ea3_hbm)
    ea_vs = (ea0_v, ea1_v, ea2_v, ea3_v)
    a_hbms = (a0_hbm, a1_hbm, a2_hbm)
    bufs = (
        (sidxA, didxA, (a0A, a1A, a2A), rlA, rrA, semLA, semGA),
        (sidxB, didxB, (a0B, a1B, a2B), rlB, rrB, semLB, semGB),
    )

    def fire_loads(bi, bs):
        sidx, didx, avs, _rl, _rr, semL, _semG = bs
        e0 = wid * ept + bi * B
        pltpu.async_copy(s_hbm.at[pl.ds(e0, B)], sidx, semL)
        pltpu.async_copy(d_hbm.at[pl.ds(e0, B)], didx, semL)
        if has_attr:
            for k in range(3):
                pltpu.async_copy(a_hbms[k].at[pl.ds(e0, B)], avs[k], semL)

    def wait_loads(bi, bs):
        sidx, didx, avs, _rl, _rr, semL, _semG = bs
        e0 = wid * ept + bi * B
        pltpu.make_async_copy(s_hbm.at[pl.ds(e0, B)], sidx, semL).wait()
        pltpu.make_async_copy(d_hbm.at[pl.ds(e0, B)], didx, semL).wait()
        if has_attr:
            for k in range(3):
                pltpu.make_async_copy(a_hbms[k].at[pl.ds(e0, B)], avs[k],
                                      semL).wait()

    def fire_gathers(bs):
        sidx, didx, _avs, rl, rr, _semL, semG = bs
        pltpu.async_copy(xl_hbm.at[sidx], rl, semG)
        pltpu.async_copy(xr_hbm.at[didx], rr, semG)

    def wait_gathers(bs):
        sidx, didx, _avs, rl, rr, _semL, semG = bs
        pltpu.make_async_copy(xl_hbm.at[sidx], rl, semG).wait()
        pltpu.make_async_copy(xr_hbm.at[didx], rr, semG).wait()

    fire_loads(0, bufs[0])
    wait_loads(0, bufs[0])
    fire_gathers(bufs[0])

    def phase(i, bs, bsn, last_guard):
        _sidx, _didx, avs, rows_l, rows_r, _semL, _semG = bs
        e0 = wid * ept + i * B
        if last_guard is None:
            fire_loads(i + 1, bsn)
        else:
            @pl.when(last_guard)
            def _fl():
                fire_loads(i + 1, bsn)
        wait_gathers(bs)

        def group(g, carry):
            attr_g = None
            if has_attr:
                attr_g = [avs[k][pl.ds(g * 16, 16)] for k in range(3)]
            att_r = [att_v[pl.ds(v * 16, 16)] for v in range(8)]
            we_r = None
            if has_attr:
                we_r = [[we_v[k, pl.ds(v * 16, 16)] for v in range(8)]
                        for k in range(3)]

            def edge(j, carry2):
                e = g * 16 + j
                if has_attr:
                    aks = [jnp.sum(jnp.where(lane == j, attr_g[k], 0.0))
                           for k in range(3)]
                cons = []
                for v in range(8):
                    s = (rows_l[e, pl.ds(v * 16, 16)]
                         + rows_r[e, pl.ds(v * 16, 16)])
                    if has_attr:
                        s = s + aks[0] * we_r[0][v] + aks[1] * we_r[1][v] \
                            + aks[2] * we_r[2][v]
                    l = jnp.maximum(s, 0.2 * s)
                    cons.append(l * att_r[v])
                out = []
                for h in range(4):
                    sh = cons[2 * h] + cons[2 * h + 1]
                    alpha = jnp.sum(sh)
                    ev = jnp.exp(jnp.full((16,), alpha, jnp.float32))
                    out.append(jnp.where(lane == j, ev, carry2[h]))
                return tuple(out)

            z = jnp.zeros((16,), jnp.float32)
            eas = lax.fori_loop(0, 16, edge, (z, z, z, z))
            for h in range(4):
                ea_vs[h][pl.ds(g * 16, 16)] = eas[h]
            return carry

        lax.fori_loop(0, B // 16, group, 0)

        def fire_next_gather():
            wait_loads(i + 1, bsn)
            fire_gathers(bsn)
        if last_guard is None:
            fire_next_gather()
        else:
            @pl.when(last_guard)
            def _fg():
                fire_next_gather()

        for h in range(4):
            pltpu.async_copy(ea_vs[h], ea_outs[h].at[pl.ds(e0, B)], semW)
        for h in range(4):
            pltpu.make_async_copy(ea_vs[h], ea_outs[h].at[pl.ds(e0, B)],
                                  semW).wait()

    def pair(p, carry):
        phase(2 * p, bufs[0], bufs[1], None)
        phase(2 * p + 1, bufs[1], bufs[0], p < n_blocks // 2 - 1)
        return carry

    lax.fori_loop(0, n_blocks // 2, pair, 0)


def _p1(xl, xr, s_idx, d_idx, attrs, We, att_flat):
    e_pad = s_idx.shape[0]
    ept = e_pad // (NC * NS)
    n_blocks = ept // B
    has_attr = attrs is not None
    mesh = plsc.VectorSubcoreMesh(core_axis_name="c", subcore_axis_name="s")
    if not has_attr:
        attrs = [jnp.zeros((128,), jnp.float32)] * 3
        We = jnp.zeros((3, D), jnp.float32)
    kern = functools.partial(
        pl.kernel,
        mesh=mesh,
        compiler_params=_SC_PARAMS,
        out_type=tuple(jax.ShapeDtypeStruct((e_pad,), jnp.float32)
                       for _ in range(4)),
        scratch_types=[
            pltpu.VMEM((B,), jnp.int32),
            pltpu.VMEM((B,), jnp.int32),
            pltpu.VMEM((B,), jnp.float32),
            pltpu.VMEM((B,), jnp.float32),
            pltpu.VMEM((B,), jnp.float32),
            pltpu.VMEM((B, D), jnp.float32),
            pltpu.VMEM((B, D), jnp.float32),
            pltpu.VMEM((B,), jnp.int32),
            pltpu.VMEM((B,), jnp.int32),
            pltpu.VMEM((B,), jnp.float32),
            pltpu.VMEM((B,), jnp.float32),
            pltpu.VMEM((B,), jnp.float32),
            pltpu.VMEM((B, D), jnp.float32),
            pltpu.VMEM((B, D), jnp.float32),
            pltpu.VMEM((3, D), jnp.float32),
            pltpu.VMEM((D,), jnp.float32),
            pltpu.VMEM((B,), jnp.float32),
            pltpu.VMEM((B,), jnp.float32),
            pltpu.VMEM((B,), jnp.float32),
            pltpu.VMEM((B,), jnp.float32),
            pltpu.SemaphoreType.DMA,
            pltpu.SemaphoreType.DMA,
            pltpu.SemaphoreType.DMA,
            pltpu.SemaphoreType.DMA,
            pltpu.SemaphoreType.DMA,
        ],
    )(functools.partial(_p1_body, has_attr, n_blocks, ept))
    return kern(xl, xr, s_idx, d_idx, attrs[0], attrs[1], attrs[2],
                We, att_flat)


def _p2_body(n_real_edges, n_blocks, ept, rc, ncper,
             xl_hbm, key_hbm, g_hbm, ea0_hbm, ea1_hbm, ea2_hbm, ea3_hbm,
             out_hbm, den_hbm,
             kidxA, sidxA, ea0A, ea1A, ea2A, ea3A, dlA, dpA, rowsA,
             kidxB, sidxB, ea0B, ea1B, ea2B, ea3B, dlB, dpB, rowsB,
             easc, zbuf, acc, dacc,
             semLA, semLB, semGA, semGB):
    ea_hbms = (ea0_hbm, ea1_hbm, ea2_hbm, ea3_hbm)
    bufs = (
        (kidxA, sidxA, (ea0A, ea1A, ea2A, ea3A), dlA, dpA, rowsA,
         semLA, semGA),
        (kidxB, sidxB, (ea0B, ea1B, ea2B, ea3B), dlB, dpB, rowsB,
         semLB, semGB),
    )
    core = lax.axis_index("c")
    sub = lax.axis_index("s")
    rpt = rc // NS
    rc8 = rc // 16
    rc8p = rc8 + 32
    lane = lax.iota(jnp.int32, 16)

    def zrow(r, _):
        for v in range(D // 16):
            zbuf[r, pl.ds(v * 16, 16)] = jnp.zeros((16,), jnp.float32)
        return _
    lax.fori_loop(0, 32, zrow, 0)

    def zrowe(r, _):
        for v in range(D // 16):
            easc[r, pl.ds(v * 16, 16)] = jnp.zeros((16,), jnp.float32)
        return _
    lax.fori_loop(0, B, zrowe, 0)

    def fire_loads(bi, bs):
        kidx, sidx, eav, _dl, _dp, _rows, semL, _semG = bs
        e0 = sub * ept + bi * B
        pltpu.async_copy(key_hbm.at[pl.ds(e0, B)], kidx, semL)
        pltpu.async_copy(g_hbm.at[pl.ds(e0, B)], sidx, semL)
        for h in range(4):
            pltpu.async_copy(ea_hbms[h].at[pl.ds(e0, B)], eav[h], semL)

    def wait_loads(bi, bs):
        kidx, sidx, eav, _dl, _dp, _rows, semL, _semG = bs
        e0 = sub * ept + bi * B
        pltpu.make_async_copy(key_hbm.at[pl.ds(e0, B)], kidx, semL).wait()
        pltpu.make_async_copy(g_hbm.at[pl.ds(e0, B)], sidx, semL).wait()
        for h in range(4):
            pltpu.make_async_copy(ea_hbms[h].at[pl.ds(e0, B)], eav[h],
                                  semL).wait()

    for cc in range(ncper):
        chunk = core * ncper + cc
        r0 = chunk * rc
        for z in range(rpt // 32):
            pltpu.sync_copy(zbuf, acc.at[pl.ds(sub * rpt + z * 32, 32)])
        for z in range(rc8p // 512):
            pltpu.sync_copy(zbuf, dacc.at[pl.ds(z * 512 + sub * 32, 32)])
        krem = (rc8p % 512) // 32

        @pl.when(sub < krem)
        def _zd():
            pltpu.sync_copy(
                zbuf, dacc.at[pl.ds((rc8p // 512) * 512 + sub * 32, 32)])
        plsc.subcore_barrier()

        fire_loads(0, bufs[0])
        wait_loads(0, bufs[0])
        pltpu.async_copy(xl_hbm.at[bufs[0][1]], bufs[0][5], bufs[0][7])

        def phase(i, bs, bsn, p, last_guard):
            kidx, sidx, eav, dl_v, dp_v, rows, _semL, semG = bs
            e0 = sub * ept + i * B
            if last_guard is None:
                fire_loads(i + 1, bsn)
            else:
                @pl.when(last_guard)
                def _fl():
                    fire_loads(i + 1, bsn)
            pltpu.make_async_copy(xl_hbm.at[sidx], rows, semG).wait()

            def group(g, carry):
                kv = kidx[pl.ds(g * 16, 16)]
                dl = kv - r0
                eg = e0 + g * 16
                m = (dl >= 0) & (dl < rc) & ((eg + lane) < n_real_edges)
                dl_v[pl.ds(g * 16, 16)] = jnp.where(m, dl, rc)
                dp_v[pl.ds(g * 16, 16)] = jnp.where(m, dl >> 4, rc8)
                fm = jnp.where(m, 1.0, 0.0)
                col = (dl & 15) * 8
                ridx = g * 16 + lane
                eah = [eav[h][pl.ds(g * 16, 16)] * fm for h in range(4)]

                @pl.when(jnp.any(m))
                def _scale():
                    for h in range(4):
                        plsc.store_scatter(easc, [ridx, col + h], eah[h])

                    def edge(j, _):
                        for u in range(4):
                            e = g * 16 + j * 4 + u
                            scs = [jnp.sum(jnp.where(lane == j * 4 + u,
                                                     eah[h], 0.0))
                                   for h in range(4)]
                            for v in range(8):
                                sc = scs[v // 2]
                                rows[e, pl.ds(v * 16, 16)] = (
                                    rows[e, pl.ds(v * 16, 16)] * sc)
                        return _
                    lax.fori_loop(0, 4, edge, 0)
                return carry

            lax.fori_loop(0, B // 16, group, 0)

            def fire_next_gather():
                wait_loads(i + 1, bsn)
                pltpu.async_copy(xl_hbm.at[bsn[1]], bsn[5], bsn[7])
            if last_guard is None:
                fire_next_gather()
            else:
                @pl.when(last_guard)
                def _fg():
                    fire_next_gather()

            pltpu.sync_copy(rows, acc.at[dl_v], add=True)
            pltpu.sync_copy(easc, dacc.at[dp_v], add=True)

            def unscatter(g, carry):
                kv = kidx[pl.ds(g * 16, 16)]
                dl = kv - r0
                eg = e0 + g * 16
                m = (dl >= 0) & (dl < rc) & ((eg + lane) < n_real_edges)
                col = (dl & 15) * 8
                ridx = g * 16 + lane
                zv = jnp.zeros((16,), jnp.float32)

                @pl.when(jnp.any(m))
                def _zs():
                    for h in range(4):
                        plsc.store_scatter(easc, [ridx, col + h], zv)
                return carry

            lax.fori_loop(0, B // 16, unscatter, 0)

        def pair(p, carry):
            phase(2 * p, bufs[0], bufs[1], p, None)
            phase(2 * p + 1, bufs[1], bufs[0], p, p < n_blocks // 2 - 1)
            return carry

        lax.fori_loop(0, n_blocks // 2, pair, 0)
        plsc.subcore_barrier()
        pltpu.sync_copy(acc.at[pl.ds(sub * rpt, rpt)],
                        out_hbm.at[pl.ds(r0 + sub * rpt, rpt)])
        for z in range(rc8 // 512):
            pltpu.sync_copy(
                dacc.at[pl.ds(z * 512 + sub * 32, 32)],
                den_hbm.at[pl.ds(chunk * rc8 + z * 512 + sub * 32, 32)])
        krem2 = (rc8 % 512) // 32

        @pl.when(sub < krem2)
        def _rd():
            off = (rc8 // 512) * 512 + sub * 32
            pltpu.sync_copy(dacc.at[pl.ds(off, 32)],
                            den_hbm.at[pl.ds(chunk * rc8 + off, 32)])
        plsc.subcore_barrier()


def _p2(xl, key_idx, gather_idx, ea, n_real_edges, n_pad, rc, ncper):
    e_pad = key_idx.shape[0]
    ept = e_pad // NS
    n_blocks = ept // B
    rc8 = rc // 16
    mesh = plsc.VectorSubcoreMesh(core_axis_name="c", subcore_axis_name="s")
    kern = functools.partial(
        pl.kernel,
        mesh=mesh,
        compiler_params=_SC_PARAMS,
        out_type=(jax.ShapeDtypeStruct((n_pad, D), jnp.float32),
                  jax.ShapeDtypeStruct((n_pad // 16, D), jnp.float32)),
        scratch_types=[
            pltpu.VMEM((B,), jnp.int32),
            pltpu.VMEM((B,), jnp.int32),
            pltpu.VMEM((B,), jnp.float32),
            pltpu.VMEM((B,), jnp.float32),
            pltpu.VMEM((B,), jnp.float32),
            pltpu.VMEM((B,), jnp.float32),
            pltpu.VMEM((B,), jnp.int32),
            pltpu.VMEM((B,), jnp.int32),
            pltpu.VMEM((B, D), jnp.float32),
            pltpu.VMEM((B,), jnp.int32),
            pltpu.VMEM((B,), jnp.int32),
            pltpu.VMEM((B,), jnp.float32),
            pltpu.VMEM((B,), jnp.float32),
            pltpu.VMEM((B,), jnp.float32),
            pltpu.VMEM((B,), jnp.float32),
            pltpu.VMEM((B,), jnp.int32),
            pltpu.VMEM((B,), jnp.int32),
            pltpu.VMEM((B, D), jnp.float32),
            pltpu.VMEM((B, D), jnp.float32),
            pltpu.VMEM((32, D), jnp.float32),
            pltpu.VMEM_SHARED((rc + 8, D), jnp.float32),
            pltpu.VMEM_SHARED((rc8 + 32, D), jnp.float32),
            pltpu.SemaphoreType.DMA,
            pltpu.SemaphoreType.DMA,
            pltpu.SemaphoreType.DMA,
            pltpu.SemaphoreType.DMA,
        ],
    )(functools.partial(_p2_body, n_real_edges, n_blocks, ept, rc, ncper))
    return kern(xl, key_idx, gather_idx, ea[0], ea[1], ea[2], ea[3])


def _fin1_body(n_real_edges,
               xl_ref, xr_ref, osc_ref, dsc_ref, asum_ref, we_ref, att_ref,
               bias_ref, wl2_ref, bl2_ref, xl2_ref):
    mean3 = asum_ref[0, 0:3] / n_real_edges
    eproj = jnp.sum(mean3[:, None] * we_ref[...], axis=0)
    xl = xl_ref[...]
    s = xl + xr_ref[...] + eproj[None, :]
    l = jnp.maximum(s, 0.2 * s)
    alpha = jnp.sum((l * att_ref[...]).reshape(-1, H, C), axis=-1)
    ea = jnp.exp(alpha)
    den = dsc_ref[:, 0:4] + ea + 1e-16
    eae = jnp.broadcast_to(ea[:, :, None], (ea.shape[0], H, C)).reshape(-1, D)
    dene = jnp.broadcast_to(den[:, :, None],
                            (den.shape[0], H, C)).reshape(-1, D)
    item_h = jnp.maximum((osc_ref[...] + eae * xl) / dene + bias_ref[...],
                         0.0)
    xl2_ref[...] = jnp.dot(item_h, wl2_ref[...],
                           preferred_element_type=jnp.float32) + bl2_ref[...]


def _fin1(n, n_pad, n_real_edges, xl1, xr1, osc, dsc, asum, We1, att1f,
          bias1, Wl2, bl2):
    grid = n // R
    return pl.pallas_call(
        functools.partial(_fin1_body, n_real_edges),
        grid=(grid,),
        in_specs=[
            pl.BlockSpec((R, D), lambda i: (i, 0)),
            pl.BlockSpec((R, D), lambda i: (i, 0)),
            pl.BlockSpec((R, D), lambda i: (i, 0)),
            pl.BlockSpec((R, 8), lambda i: (i, 0)),
            pl.BlockSpec((1, D), lambda i: (0, 0)),
            pl.BlockSpec((3, D), lambda i: (0, 0)),
            pl.BlockSpec((1, D), lambda i: (0, 0)),
            pl.BlockSpec((1, D), lambda i: (0, 0)),
            pl.BlockSpec((D, D), lambda i: (0, 0)),
            pl.BlockSpec((1, D), lambda i: (0, 0)),
        ],
        out_specs=pl.BlockSpec((R, D), lambda i: (i, 0)),
        out_shape=jax.ShapeDtypeStruct((n, D), jnp.float32),
    )(xl1, xr1, osc, dsc, asum, We1, att1f, bias1, Wl2, bl2.reshape(1, D))


def _fin2_body(xl_ref, xr_ref, osc_ref, dsc_ref, att_ref, bias_ref,
               wp1_ref, bp1_ref, wp2_ref, bp2_ref, uh_ref, z_ref):
    xl = xl_ref[...]
    s = xl + xr_ref[...]
    l = jnp.maximum(s, 0.2 * s)
    alpha = jnp.sum((l * att_ref[...]).reshape(-1, H, C), axis=-1)
    ea = jnp.exp(alpha)
    den = dsc_ref[:, 0:4] + ea + 1e-16
    eae = jnp.broadcast_to(ea[:, :, None], (ea.shape[0], H, C)).reshape(-1, D)
    dene = jnp.broadcast_to(den[:, :, None],
                            (den.shape[0], H, C)).reshape(-1, D)
    uh = (osc_ref[...] + eae * xl) / dene + bias_ref[...]
    uh_ref[...] = uh
    hh = jnp.maximum(
        jnp.dot(uh, wp1_ref[...], preferred_element_type=jnp.float32)
        + bp1_ref[...], 0.0)
    z_ref[...] = jnp.dot(hh, wp2_ref[...],
                         preferred_element_type=jnp.float32) + bp2_ref[...]


def _fin2(n, xl2, xr2, osc, dsc, att2f, bias2, Wp1, bp1, Wp2, bp2):
    grid = n // R
    return pl.pallas_call(
        _fin2_body,
        grid=(grid,),
        in_specs=[
            pl.BlockSpec((R, D), lambda i: (i, 0)),
            pl.BlockSpec((R, D), lambda i: (i, 0)),
            pl.BlockSpec((R, D), lambda i: (i, 0)),
            pl.BlockSpec((R, 8), lambda i: (i, 0)),
            pl.BlockSpec((1, D), lambda i: (0, 0)),
            pl.BlockSpec((1, D), lambda i: (0, 0)),
            pl.BlockSpec((D, D), lambda i: (0, 0)),
            pl.BlockSpec((1, D), lambda i: (0, 0)),
            pl.BlockSpec((D, D), lambda i: (0, 0)),
            pl.BlockSpec((1, D), lambda i: (0, 0)),
        ],
        out_specs=[
            pl.BlockSpec((R, D), lambda i: (i, 0)),
            pl.BlockSpec((R, D), lambda i: (i, 0)),
        ],
        out_shape=[jax.ShapeDtypeStruct((n, D), jnp.float32)] * 2,
    )(xl2, xr2, osc, dsc, att2f, bias2, Wp1, bp1.reshape(1, D), Wp2,
      bp2.reshape(1, D))


def kernel(customer_x, fund_x, edge_index, edge_attr, Wu, bu, Wi, bi,
           Wl1, bl1, Wr1, br1, att1, We1, bias1,
           Wl2, bl2, Wr2, br2, att2, bias2, Wp1, bp1, Wp2, bp2):
    n = customer_x.shape[0]
    e = edge_index.shape[1]
    e_pad = _cdiv(e, NC * NS * B) * NC * NS * B
    ncper = 3
    rc = _cdiv(_cdiv(n, 2 * ncper), NS * 32) * NS * 32
    n_pad = 2 * ncper * rc

    src = edge_index[0]
    dst = edge_index[1]
    srcp = jnp.zeros((e_pad,), jnp.int32).at[:e].set(src)
    dstp = jnp.zeros((e_pad,), jnp.int32).at[:e].set(dst)
    attrT = jnp.zeros((3, e_pad), jnp.float32).at[:, :e].set(edge_attr.T)
    attrs = [attrT[0], attrT[1], attrT[2]]
    att1f = att1.reshape(1, D)
    att2f = att2.reshape(1, D)

    xl1, xr1, xr2 = _prep(customer_x, Wu, bu, fund_x, Wi, bi,
                          Wl1, bl1, Wr1, br1, Wr2, br2)
    asum = _attrsum(attrT)

    ea1 = _p1(xl1, xr1, srcp, dstp, attrs, We1, att1f.reshape(D))
    osc1, dp1 = _p2(xl1, dstp, srcp, ea1, e, n_pad, rc, ncper)
    dsc1 = dp1.reshape(n_pad, 8)
    xl2 = _fin1(n, n_pad, e, xl1, xr1, osc1, dsc1, asum, We1, att1f,
                bias1.reshape(1, D), Wl2, bl2)

    ea2 = _p1(xl2, xr2, dstp, srcp, None, None, att2f.reshape(D))
    osc2, dp2 = _p2(xl2, srcp, dstp, ea2, e, n_pad, rc, ncper)
    dsc2 = dp2.reshape(n_pad, 8)
    user_h, z = _fin2(n, xl2, xr2, osc2, dsc2, att2f, bias2.reshape(1, D),
                      Wp1, bp1, Wp2, bp2)
    return (user_h, z)
